# Initial kernel scaffold; baseline (speedup 1.0000x reference)
import jax
import jax.numpy as jnp
from jax.experimental import pallas as pl


def kernel(preds):
    raise NotImplementedError("write your pallas kernel here")


# speedup vs baseline: 9.9362x; 9.9362x over previous
import functools

import jax
import jax.numpy as jnp
from jax.experimental import pallas as pl
from jax.experimental.pallas import tpu as pltpu

_ROWS = 2048
_COLS = 8192
_BLOCK_COLS = 512
_NBLK = _COLS // _BLOCK_COLS
_NEG = float("-inf")


def _shift_up(a, k):
    return jnp.concatenate(
        [a[k:, :], jnp.full((k, a.shape[1]), _NEG, a.dtype)], axis=0
    )


def _sliding_max45(x):
    rows, cols = x.shape
    xp = jnp.concatenate([jnp.full((22, cols), _NEG, x.dtype), x], axis=0)
    f = xp
    for k in (1, 2, 4, 8, 16):
        f = jnp.maximum(f, _shift_up(f, k))
    f45 = jnp.maximum(f, _shift_up(f, 13))
    return f45[:rows, :]


def _max_kernel(x_ref, o_ref):
    o_ref[0, 0, 0] = jnp.max(x_ref[...])


def _nms_kernel(bmax_ref, x_ref, o_ref):
    gmax = bmax_ref[0, 0, 0]
    for i in range(1, _NBLK):
        gmax = jnp.maximum(gmax, bmax_ref[i, 0, 0])
    thresh = gmax * jnp.float32(0.5)
    x = x_ref[...]
    m = _sliding_max45(x)
    keep = (x == m) & (x >= thresh)
    o_ref[...] = jnp.where(keep, x, jnp.float32(0.0))


@jax.jit
def kernel(preds):
    bmax = pl.pallas_call(
        _max_kernel,
        grid=(_NBLK,),
        in_specs=[
            pl.BlockSpec((_ROWS, _BLOCK_COLS), lambda i: (0, i)),
        ],
        out_specs=pl.BlockSpec(
            (1, 1, 1), lambda i: (i, 0, 0), memory_space=pltpu.SMEM
        ),
        out_shape=jax.ShapeDtypeStruct((_NBLK, 1, 1), jnp.float32),
        compiler_params=pltpu.CompilerParams(
            dimension_semantics=("arbitrary",),
        ),
    )(preds)

    out = pl.pallas_call(
        _nms_kernel,
        grid=(_NBLK,),
        in_specs=[
            pl.BlockSpec(memory_space=pltpu.SMEM),
            pl.BlockSpec((_ROWS, _BLOCK_COLS), lambda i: (0, i)),
        ],
        out_specs=pl.BlockSpec((_ROWS, _BLOCK_COLS), lambda i: (0, i)),
        out_shape=jax.ShapeDtypeStruct((_ROWS, _COLS), jnp.float32),
        compiler_params=pltpu.CompilerParams(
            dimension_semantics=("arbitrary",),
        ),
    )(bmax, preds)
    return out
